# Initial kernel scaffold; baseline (speedup 1.0000x reference)
import jax
import jax.numpy as jnp
from jax.experimental import pallas as pl


def kernel(x, edge_index, W1, b1, W2, b2, W3, b3):
    raise NotImplementedError("write your pallas kernel here")


# speedup vs baseline: 22.5650x; 22.5650x over previous
import jax
import jax.numpy as jnp
from jax import lax
from jax.experimental import pallas as pl
from jax.experimental.pallas import tpu as pltpu
from jax.experimental.pallas import tpu_sc as plsc

N = 10000
NP = 10240
E = 320000
EP = 327680
NC = 2
NS = 16
RPT = NP // NS
C = 128
K = EP // (NC * NS * C)
DEG_W = 16
R = 1024


def _sc_mesh():
    return plsc.VectorSubcoreMesh(core_axis_name="c", subcore_axis_name="s")


def _deg_body(dst_hbm, ones_hbm, zeros_hbm, out_hbm, acc_sh, ones_v, dst_v, sem):
    c = lax.axis_index("c")
    s = lax.axis_index("s")
    r0 = s * RPT

    pltpu.sync_copy(ones_hbm, ones_v)
    pltpu.sync_copy(zeros_hbm.at[pl.ds(r0, RPT)], acc_sh.at[pl.ds(r0, RPT)])
    rowbase = (c * NS + s) * K
    plsc.subcore_barrier()

    def body(j, carry):
        pltpu.sync_copy(dst_hbm.at[rowbase + j], dst_v)
        pltpu.async_copy(ones_v, acc_sh.at[dst_v], sem, add=True).wait()
        return carry

    lax.fori_loop(0, K, body, 0)
    plsc.subcore_barrier()
    pltpu.sync_copy(acc_sh.at[pl.ds(r0, RPT)],
                    out_hbm.at[pl.ds(c * NP + r0, RPT)])


def _deg_call(dst2d, zeros16):
    ones = jnp.ones((C, DEG_W), jnp.float32)
    kern = pl.kernel(
        _deg_body,
        out_type=jax.ShapeDtypeStruct((NC * NP, DEG_W), jnp.float32),
        mesh=_sc_mesh(),
        compiler_params=pltpu.CompilerParams(use_tc_tiling_on_sc=False),
        scratch_types=[
            pltpu.VMEM_SHARED((NP, DEG_W), jnp.float32),
            pltpu.VMEM((C, DEG_W), jnp.float32),
            pltpu.VMEM((C,), jnp.int32),
            pltpu.SemaphoreType.DMA,
        ],
        name="gcn_degree_sc",
    )
    return kern(dst2d, ones, zeros16)


def _scatter_body(hs_hbm, src_hbm, dst_hbm, zeros_hbm, out_hbm,
                  table_sh, acc_sh, rows_v, src_v, dst_v, sem):
    c = lax.axis_index("c")
    s = lax.axis_index("s")
    r0 = s * RPT
    pltpu.sync_copy(hs_hbm.at[pl.ds(r0, RPT)], table_sh.at[pl.ds(r0, RPT)])
    pltpu.sync_copy(zeros_hbm.at[pl.ds(r0, RPT)], acc_sh.at[pl.ds(r0, RPT)])
    rowbase = (c * NS + s) * K
    plsc.subcore_barrier()

    def body(j, carry):
        pltpu.sync_copy(src_hbm.at[rowbase + j], src_v)
        pltpu.sync_copy(dst_hbm.at[rowbase + j], dst_v)
        pltpu.async_copy(table_sh.at[src_v], rows_v, sem).wait()
        pltpu.sync_copy(rows_v, acc_sh.at[dst_v], add=True)
        return carry

    lax.fori_loop(0, K, body, 0)
    plsc.subcore_barrier()
    pltpu.sync_copy(acc_sh.at[pl.ds(r0, RPT)],
                    out_hbm.at[pl.ds(c * NP + r0, RPT)])


def _scatter_call(hs, src2d, dst2d, zeros64):
    kern = pl.kernel(
        _scatter_body,
        out_type=jax.ShapeDtypeStruct((NC * NP, 64), jnp.float32),
        mesh=_sc_mesh(),
        compiler_params=pltpu.CompilerParams(use_tc_tiling_on_sc=False),
        scratch_types=[
            pltpu.VMEM_SHARED((NP, 64), jnp.float32),
            pltpu.VMEM_SHARED((NP, 64), jnp.float32),
            pltpu.VMEM((C, 64), jnp.float32),
            pltpu.VMEM((C,), jnp.int32),
            pltpu.VMEM((C,), jnp.int32),
            pltpu.SemaphoreType.DMA,
        ],
        name="gcn_edge_scatter_sc",
    )
    return kern(hs, src2d, dst2d, zeros64)


def _tc_a_body(dp0_ref, dp1_ref, x_ref, w_ref, h1s_ref, dis_ref):
    deg = 1.0 + dp0_ref[:, 0:1] + dp1_ref[:, 0:1]
    dis = lax.rsqrt(deg)
    h = jnp.dot(x_ref[:], w_ref[:], preferred_element_type=jnp.float32)
    h1s_ref[:] = h * dis
    dis_ref[:] = dis


def _tc_a(dp0, dp1, x, w1):
    return pl.pallas_call(
        _tc_a_body,
        grid=(NP // R,),
        in_specs=[
            pl.BlockSpec((R, DEG_W), lambda i: (i, 0)),
            pl.BlockSpec((R, DEG_W), lambda i: (i, 0)),
            pl.BlockSpec((R, 128), lambda i: (i, 0)),
            pl.BlockSpec((128, 64), lambda i: (0, 0)),
        ],
        out_specs=[
            pl.BlockSpec((R, 64), lambda i: (i, 0)),
            pl.BlockSpec((R, 1), lambda i: (i, 0)),
        ],
        out_shape=[
            jax.ShapeDtypeStruct((NP, 64), jnp.float32),
            jax.ShapeDtypeStruct((NP, 1), jnp.float32),
        ],
        name="gcn_l1_matmul_tc",
    )(dp0, dp1, x, w1)


def _tc_b_body(h1s_ref, p0_ref, p1_ref, dis_ref, b1_ref, w_ref, out_ref):
    agg = h1s_ref[:] + p0_ref[:] + p1_ref[:]
    z = jnp.maximum(agg * dis_ref[:] + b1_ref[:], 0.0)
    out_ref[:] = jnp.dot(z, w_ref[:], preferred_element_type=jnp.float32) * dis_ref[:]


def _tc_b(h1s, p0, p1, dis, b1, w23):
    return pl.pallas_call(
        _tc_b_body,
        grid=(NP // R,),
        in_specs=[
            pl.BlockSpec((R, 64), lambda i: (i, 0)),
            pl.BlockSpec((R, 64), lambda i: (i, 0)),
            pl.BlockSpec((R, 64), lambda i: (i, 0)),
            pl.BlockSpec((R, 1), lambda i: (i, 0)),
            pl.BlockSpec((1, 64), lambda i: (0, 0)),
            pl.BlockSpec((64, 64), lambda i: (0, 0)),
        ],
        out_specs=pl.BlockSpec((R, 64), lambda i: (i, 0)),
        out_shape=jax.ShapeDtypeStruct((NP, 64), jnp.float32),
        name="gcn_l23_matmul_tc",
    )(h1s, p0, p1, dis, b1, w23)


def _tc_c_body(h23s_ref, q0_ref, q1_ref, dis_ref, b23_ref, out_ref):
    agg = h23s_ref[:] + q0_ref[:] + q1_ref[:]
    out_ref[:] = agg * dis_ref[:] + b23_ref[:]


def _tc_c(h23s, q0, q1, dis, b23):
    return pl.pallas_call(
        _tc_c_body,
        grid=(NP // R,),
        in_specs=[
            pl.BlockSpec((R, 64), lambda i: (i, 0)),
            pl.BlockSpec((R, 64), lambda i: (i, 0)),
            pl.BlockSpec((R, 64), lambda i: (i, 0)),
            pl.BlockSpec((R, 1), lambda i: (i, 0)),
            pl.BlockSpec((1, 64), lambda i: (0, 0)),
        ],
        out_specs=pl.BlockSpec((R, 64), lambda i: (i, 0)),
        out_shape=jax.ShapeDtypeStruct((NP, 64), jnp.float32),
        name="gcn_out_scale_tc",
    )(h23s, q0, q1, dis, b23)


def kernel(x, edge_index, W1, b1, W2, b2, W3, b3):
    pad = jnp.full((EP - E,), N, jnp.int32)
    src2d = jnp.concatenate([edge_index[0].astype(jnp.int32), pad]).reshape(-1, C)
    dst2d = jnp.concatenate([edge_index[1].astype(jnp.int32), pad]).reshape(-1, C)
    zeros16 = jnp.zeros((NP, DEG_W), jnp.float32)
    zeros64 = jnp.zeros((NP, 64), jnp.float32)
    xp = jnp.zeros((NP, 128), jnp.float32).at[:N].set(x)
    w23 = jnp.concatenate([W2, W3], axis=1)
    b23 = jnp.concatenate([b2, b3])[None, :]
    b1r = b1[None, :]

    degp = _deg_call(dst2d, zeros16)
    h1s, dis = _tc_a(degp[:NP], degp[NP:], xp, W1)
    sp = _scatter_call(h1s, src2d, dst2d, zeros64)
    h23s = _tc_b(h1s, sp[:NP], sp[NP:], dis, b1r, w23)
    qp = _scatter_call(h23s, src2d, dst2d, zeros64)
    out = _tc_c(h23s, qp[:NP], qp[NP:], dis, b23)
    return out[:N, :32], out[:N, 32:]


# speedup vs baseline: 36.0542x; 1.5978x over previous
import jax
import jax.numpy as jnp
from jax import lax
from jax.experimental import pallas as pl
from jax.experimental.pallas import tpu as pltpu
from jax.experimental.pallas import tpu_sc as plsc

N = 10000
NP = 10240
E = 320000
EP = 327680
NC = 2
NS = 16
RPT = NP // NS
C = 128
K = EP // (NC * NS * C)
DEG_W = 16
R = 1024


def _sc_mesh():
    return plsc.VectorSubcoreMesh(core_axis_name="c", subcore_axis_name="s")


def _deg_body(dst_hbm, ones_hbm, zeros_hbm, out_hbm, acc_sh, ones_v, dst_v, sem):
    c = lax.axis_index("c")
    s = lax.axis_index("s")
    r0 = s * RPT

    pltpu.sync_copy(ones_hbm, ones_v)
    pltpu.sync_copy(zeros_hbm.at[pl.ds(r0, RPT)], acc_sh.at[pl.ds(r0, RPT)])
    rowbase = (c * NS + s) * K
    pltpu.sync_copy(dst_hbm.at[pl.ds(rowbase, K)], dst_v)
    plsc.subcore_barrier()

    def body(j, carry):
        pltpu.sync_copy(ones_v, acc_sh.at[dst_v.at[j]], add=True)
        return carry

    lax.fori_loop(0, K, body, 0)
    plsc.subcore_barrier()
    pltpu.sync_copy(acc_sh.at[pl.ds(r0, RPT)],
                    out_hbm.at[pl.ds(c * NP + r0, RPT)])


def _deg_call(dst2d, zeros16):
    ones = jnp.ones((C, DEG_W), jnp.float32)
    kern = pl.kernel(
        _deg_body,
        out_type=jax.ShapeDtypeStruct((NC * NP, DEG_W), jnp.float32),
        mesh=_sc_mesh(),
        compiler_params=pltpu.CompilerParams(use_tc_tiling_on_sc=False),
        scratch_types=[
            pltpu.VMEM_SHARED((NP, DEG_W), jnp.float32),
            pltpu.VMEM((C, DEG_W), jnp.float32),
            pltpu.VMEM((K, C), jnp.int32),
            pltpu.SemaphoreType.DMA,
        ],
        name="gcn_degree_sc",
    )
    return kern(dst2d, ones, zeros16)


def _scatter_body(hs_hbm, src_hbm, dst_hbm, zeros_hbm, out_hbm,
                  table_sh, acc_sh, rows0_v, rows1_v, src_v, dst_v, sem0, sem1):
    c = lax.axis_index("c")
    s = lax.axis_index("s")
    r0 = s * RPT
    pltpu.sync_copy(hs_hbm.at[pl.ds(r0, RPT)], table_sh.at[pl.ds(r0, RPT)])
    pltpu.sync_copy(zeros_hbm.at[pl.ds(r0, RPT)], acc_sh.at[pl.ds(r0, RPT)])
    rowbase = (c * NS + s) * K
    pltpu.sync_copy(src_hbm.at[pl.ds(rowbase, K)], src_v)
    pltpu.sync_copy(dst_hbm.at[pl.ds(rowbase, K)], dst_v)
    plsc.subcore_barrier()

    pltpu.async_copy(table_sh.at[src_v.at[0]], rows0_v, sem0)

    def body(t, carry):
        j = 2 * t
        pltpu.async_copy(table_sh.at[src_v.at[j + 1]], rows1_v, sem1)
        pltpu.make_async_copy(table_sh.at[src_v.at[j]], rows0_v, sem0).wait()
        pltpu.sync_copy(rows0_v, acc_sh.at[dst_v.at[j]], add=True)

        @pl.when(t + 1 < K // 2)
        def _():
            pltpu.async_copy(table_sh.at[src_v.at[j + 2]], rows0_v, sem0)

        pltpu.make_async_copy(table_sh.at[src_v.at[j + 1]], rows1_v, sem1).wait()
        pltpu.sync_copy(rows1_v, acc_sh.at[dst_v.at[j + 1]], add=True)
        return carry

    lax.fori_loop(0, K // 2, body, 0)
    plsc.subcore_barrier()
    pltpu.sync_copy(acc_sh.at[pl.ds(r0, RPT)],
                    out_hbm.at[pl.ds(c * NP + r0, RPT)])


def _scatter_call(hs, src2d, dst2d, zeros64):
    kern = pl.kernel(
        _scatter_body,
        out_type=jax.ShapeDtypeStruct((NC * NP, 64), jnp.float32),
        mesh=_sc_mesh(),
        compiler_params=pltpu.CompilerParams(use_tc_tiling_on_sc=False),
        scratch_types=[
            pltpu.VMEM_SHARED((NP, 64), jnp.float32),
            pltpu.VMEM_SHARED((NP, 64), jnp.float32),
            pltpu.VMEM((C, 64), jnp.float32),
            pltpu.VMEM((C, 64), jnp.float32),
            pltpu.VMEM((K, C), jnp.int32),
            pltpu.VMEM((K, C), jnp.int32),
            pltpu.SemaphoreType.DMA,
            pltpu.SemaphoreType.DMA,
        ],
        name="gcn_edge_scatter_sc",
    )
    return kern(hs, src2d, dst2d, zeros64)


def _tc_a_body(dp0_ref, dp1_ref, x_ref, w_ref, h1s_ref, dis_ref):
    deg = 1.0 + dp0_ref[:, 0:1] + dp1_ref[:, 0:1]
    dis = lax.rsqrt(deg)
    h = jnp.dot(x_ref[:], w_ref[:], preferred_element_type=jnp.float32)
    h1s_ref[:] = h * dis
    dis_ref[:] = dis


def _tc_a(dp0, dp1, x, w1):
    return pl.pallas_call(
        _tc_a_body,
        grid=(NP // R,),
        in_specs=[
            pl.BlockSpec((R, DEG_W), lambda i: (i, 0)),
            pl.BlockSpec((R, DEG_W), lambda i: (i, 0)),
            pl.BlockSpec((R, 128), lambda i: (i, 0)),
            pl.BlockSpec((128, 64), lambda i: (0, 0)),
        ],
        out_specs=[
            pl.BlockSpec((R, 64), lambda i: (i, 0)),
            pl.BlockSpec((R, 1), lambda i: (i, 0)),
        ],
        out_shape=[
            jax.ShapeDtypeStruct((NP, 64), jnp.float32),
            jax.ShapeDtypeStruct((NP, 1), jnp.float32),
        ],
        name="gcn_l1_matmul_tc",
    )(dp0, dp1, x, w1)


def _tc_b_body(h1s_ref, p0_ref, p1_ref, dis_ref, b1_ref, w_ref, out_ref):
    agg = h1s_ref[:] + p0_ref[:] + p1_ref[:]
    z = jnp.maximum(agg * dis_ref[:] + b1_ref[:], 0.0)
    out_ref[:] = jnp.dot(z, w_ref[:], preferred_element_type=jnp.float32) * dis_ref[:]


def _tc_b(h1s, p0, p1, dis, b1, w23):
    return pl.pallas_call(
        _tc_b_body,
        grid=(NP // R,),
        in_specs=[
            pl.BlockSpec((R, 64), lambda i: (i, 0)),
            pl.BlockSpec((R, 64), lambda i: (i, 0)),
            pl.BlockSpec((R, 64), lambda i: (i, 0)),
            pl.BlockSpec((R, 1), lambda i: (i, 0)),
            pl.BlockSpec((1, 64), lambda i: (0, 0)),
            pl.BlockSpec((64, 64), lambda i: (0, 0)),
        ],
        out_specs=pl.BlockSpec((R, 64), lambda i: (i, 0)),
        out_shape=jax.ShapeDtypeStruct((NP, 64), jnp.float32),
        name="gcn_l23_matmul_tc",
    )(h1s, p0, p1, dis, b1, w23)


def _tc_c_body(h23s_ref, q0_ref, q1_ref, dis_ref, b23_ref, out_ref):
    agg = h23s_ref[:] + q0_ref[:] + q1_ref[:]
    out_ref[:] = agg * dis_ref[:] + b23_ref[:]


def _tc_c(h23s, q0, q1, dis, b23):
    return pl.pallas_call(
        _tc_c_body,
        grid=(NP // R,),
        in_specs=[
            pl.BlockSpec((R, 64), lambda i: (i, 0)),
            pl.BlockSpec((R, 64), lambda i: (i, 0)),
            pl.BlockSpec((R, 64), lambda i: (i, 0)),
            pl.BlockSpec((R, 1), lambda i: (i, 0)),
            pl.BlockSpec((1, 64), lambda i: (0, 0)),
        ],
        out_specs=pl.BlockSpec((R, 64), lambda i: (i, 0)),
        out_shape=jax.ShapeDtypeStruct((NP, 64), jnp.float32),
        name="gcn_out_scale_tc",
    )(h23s, q0, q1, dis, b23)


def kernel(x, edge_index, W1, b1, W2, b2, W3, b3):
    pad = jnp.full((EP - E,), N, jnp.int32)
    src2d = jnp.concatenate([edge_index[0].astype(jnp.int32), pad]).reshape(-1, C)
    dst2d = jnp.concatenate([edge_index[1].astype(jnp.int32), pad]).reshape(-1, C)
    zeros16 = jnp.zeros((NP, DEG_W), jnp.float32)
    zeros64 = jnp.zeros((NP, 64), jnp.float32)
    xp = jnp.zeros((NP, 128), jnp.float32).at[:N].set(x)
    w23 = jnp.concatenate([W2, W3], axis=1)
    b23 = jnp.concatenate([b2, b3])[None, :]
    b1r = b1[None, :]

    degp = _deg_call(dst2d, zeros16)
    h1s, dis = _tc_a(degp[:NP], degp[NP:], xp, W1)
    sp = _scatter_call(h1s, src2d, dst2d, zeros64)
    h23s = _tc_b(h1s, sp[:NP], sp[NP:], dis, b1r, w23)
    qp = _scatter_call(h23s, src2d, dst2d, zeros64)
    out = _tc_c(h23s, qp[:NP], qp[NP:], dis, b23)
    return out[:N, :32], out[:N, 32:]
